# baseline (device time: 59415 ns/iter reference)
import functools

import jax
import jax.numpy as jnp
from jax import lax
from jax.experimental import pallas as pl
from jax.experimental.pallas import tpu as pltpu

N_DEV = 8
WIN = 128
KV_KEEP = 384

PHASES = [
    [(0, 0, 4), (1, 1, 5)],
    [(0, 0, 3), (0, 4, 7), (1, 1, 2), (1, 5, 6)],
    [(0, 0, 1), (0, 3, 2), (0, 4, 5), (0, 7, 6),
     (1, 1, 0), (1, 2, 3), (1, 5, 4), (1, 6, 7)],
]

PEERS = {d: set() for d in range(N_DEV)}
for _phase in PHASES:
    for _c, _s, _d in _phase:
        PEERS[_s].add(_d)
        PEERS[_d].add(_s)
PEERS = {d: sorted(p) for d, p in PEERS.items()}


def kernel(x, Wq, K_ext, V_ext, Wo):
    B, Sq, Dm = x.shape
    _, Skv_sh, Hq, Dh = K_ext.shape
    Dout = Wo.shape[1]
    half = Skv_sh // 2

    def body(x_ref, wq_ref, k_ref, v_ref, wo_ref, out_ref,
             k0, v0, k1, v1, ssem, rsem):
        my = lax.axis_index("i")

        barrier = pltpu.get_barrier_semaphore()
        for d in range(N_DEV):
            @pl.when(my == d)
            def _(d=d):
                for p in PEERS[d]:
                    pl.semaphore_signal(
                        barrier, inc=1, device_id=(p,),
                        device_id_type=pl.DeviceIdType.MESH,
                    )
                pl.semaphore_wait(barrier, len(PEERS[d]))

        @pl.when(my == 0)
        def _():
            k0[...] = k_ref[...].astype(jnp.bfloat16)
            v0[...] = v_ref[...].astype(jnp.bfloat16)

        @pl.when(my == 1)
        def _():
            k1[...] = k_ref[:, :half].astype(jnp.bfloat16)
            v1[...] = v_ref[:, :half].astype(jnp.bfloat16)

        bufs = {0: (k0, v0), 1: (k1, v1)}

        def mk(buf, sem_i, dev):
            return pltpu.make_async_remote_copy(
                src_ref=buf, dst_ref=buf,
                send_sem=ssem.at[sem_i], recv_sem=rsem.at[sem_i],
                device_id=(dev,), device_id_type=pl.DeviceIdType.MESH,
            )

        for phase in PHASES:
            for chunk, src, dst in phase:
                kbuf, vbuf = bufs[chunk]

                @pl.when(my == src)
                def _(kbuf=kbuf, vbuf=vbuf, chunk=chunk, dst=dst):
                    rk = mk(kbuf, 2 * chunk, dst)
                    rv = mk(vbuf, 2 * chunk + 1, dst)
                    rk.start()
                    rv.start()
                    rk.wait_send()
                    rv.wait_send()

                @pl.when(my == dst)
                def _(kbuf=kbuf, vbuf=vbuf, chunk=chunk, src=src):
                    rk = mk(kbuf, 2 * chunk, src)
                    rv = mk(vbuf, 2 * chunk + 1, src)
                    rk.wait_recv()
                    rv.wait_recv()

        qi = lax.broadcasted_iota(jnp.int32, (Sq, KV_KEEP), 0)
        ki = lax.broadcasted_iota(jnp.int32, (Sq, KV_KEEP), 1)
        mask = jnp.abs(qi - ki) <= WIN

        wq = wq_ref[...].astype(jnp.bfloat16)
        wo = wo_ref[...].astype(jnp.bfloat16)

        for b in range(B):
            xb = x_ref[b].astype(jnp.bfloat16)
            qb = jnp.dot(
                xb, wq, preferred_element_type=jnp.float32
            ).astype(jnp.bfloat16)
            heads = []
            for h in range(Hq):
                q = qb[:, h * Dh:(h + 1) * Dh]
                kc = jnp.concatenate(
                    [k0[b, :, h, :], k1[b, :, h, :]], axis=0
                )
                vc = jnp.concatenate(
                    [v0[b, :, h, :], v1[b, :, h, :]], axis=0
                )
                s = lax.dot_general(
                    q, kc, (((1,), (1,)), ((), ())),
                    preferred_element_type=jnp.float32,
                ) * 0.125
                s = jnp.where(mask, s, -1e9)
                m = jnp.max(s, axis=1, keepdims=True)
                p = jnp.exp(s - m)
                l = jnp.sum(p, axis=1, keepdims=True)
                pn = (p / l).astype(jnp.bfloat16)
                ctx = lax.dot_general(
                    pn, vc, (((1,), (0,)), ((), ())),
                    preferred_element_type=jnp.float32,
                )
                heads.append(ctx.astype(jnp.bfloat16))
            ctx_b = jnp.concatenate(heads, axis=1)
            out_ref[b] = jnp.dot(
                ctx_b, wo, preferred_element_type=jnp.float32
            )

        @functools.partial(
            pl.run_scoped, second_barrier=pltpu.SemaphoreType.REGULAR
        )
        def _(second_barrier):
            for d in range(N_DEV):
                @pl.when(my == d)
                def _(d=d):
                    for p in PEERS[d]:
                        pl.semaphore_signal(
                            second_barrier, inc=1, device_id=(p,),
                            device_id_type=pl.DeviceIdType.MESH,
                        )
                    pl.semaphore_wait(second_barrier, len(PEERS[d]))

    out_shape = jax.ShapeDtypeStruct((B, Sq, Dout), jnp.float32)
    return pl.pallas_call(
        body,
        out_shape=out_shape,
        in_specs=[pl.BlockSpec(memory_space=pltpu.VMEM)] * 5,
        out_specs=pl.BlockSpec(memory_space=pltpu.VMEM),
        scratch_shapes=[
            pltpu.VMEM((B, Skv_sh, Hq, Dh), jnp.bfloat16),
            pltpu.VMEM((B, Skv_sh, Hq, Dh), jnp.bfloat16),
            pltpu.VMEM((B, half, Hq, Dh), jnp.bfloat16),
            pltpu.VMEM((B, half, Hq, Dh), jnp.bfloat16),
            pltpu.SemaphoreType.DMA((4,)),
            pltpu.SemaphoreType.DMA((4,)),
        ],
        compiler_params=pltpu.CompilerParams(collective_id=0),
    )(x, Wq, K_ext, V_ext, Wo)


# device time: 52935 ns/iter; 1.1224x vs baseline; 1.1224x over previous
import functools
import os

import jax
import jax.numpy as jnp
from jax import lax
from jax.experimental import pallas as pl
from jax.experimental.pallas import tpu as pltpu

N_DEV = 8
WIN = 128
try:
    _FLAGS = set(
        open(os.path.join(os.path.dirname(__file__), "exp_flags.txt"))
        .read()
        .split()
    )
except OSError:
    _FLAGS = set()
_SKIP_COMM = "skip_comm" in _FLAGS or os.environ.get("KERNEL_SKIP_COMM") == "1"
_SKIP_COMPUTE = (
    "skip_compute" in _FLAGS or os.environ.get("KERNEL_SKIP_COMPUTE") == "1"
)
KV_KEEP = 384

PHASES = [
    [(0, 0, 4), (1, 1, 5)],
    [(0, 0, 3), (0, 4, 7), (1, 1, 2), (1, 5, 6)],
    [(0, 0, 1), (0, 3, 2), (0, 4, 5), (0, 7, 6),
     (1, 1, 0), (1, 2, 3), (1, 5, 4), (1, 6, 7)],
]

PEERS = {d: set() for d in range(N_DEV)}
for _phase in PHASES:
    for _c, _s, _d in _phase:
        PEERS[_s].add(_d)
        PEERS[_d].add(_s)
PEERS = {d: sorted(p) for d, p in PEERS.items()}


def kernel(x, Wq, K_ext, V_ext, Wo):
    B, Sq, Dm = x.shape
    _, Skv_sh, Hq, Dh = K_ext.shape
    Dout = Wo.shape[1]
    half = Skv_sh // 2

    def body(x_ref, wq_ref, k_ref, v_ref, wo_ref, out_ref,
             k0, v0, k1, v1, ssem, rsem):
        my = lax.axis_index("i")

        if not _SKIP_COMM:
            _do_comm(my, x_ref, k_ref, v_ref, k0, v0, k1, v1, ssem, rsem)
        if not _SKIP_COMPUTE:
            _do_compute(x_ref, wq_ref, wo_ref, out_ref, k0, v0, k1, v1)
        if not _SKIP_COMM:
            _do_exit_barrier(my)

    def _do_comm(my, x_ref, k_ref, v_ref, k0, v0, k1, v1, ssem, rsem):
        barrier = pltpu.get_barrier_semaphore()
        for d in range(N_DEV):
            @pl.when(my == d)
            def _(d=d):
                for p in PEERS[d]:
                    pl.semaphore_signal(
                        barrier, inc=1, device_id=(p,),
                        device_id_type=pl.DeviceIdType.MESH,
                    )
                pl.semaphore_wait(barrier, len(PEERS[d]))

        @pl.when(my == 0)
        def _():
            k0[...] = k_ref[...].astype(jnp.bfloat16)
            v0[...] = v_ref[...].astype(jnp.bfloat16)

        @pl.when(my == 1)
        def _():
            k1[...] = k_ref[:, :half].astype(jnp.bfloat16)
            v1[...] = v_ref[:, :half].astype(jnp.bfloat16)

        bufs = {0: (k0, v0), 1: (k1, v1)}

        def mk(buf, sem_i, dev):
            return pltpu.make_async_remote_copy(
                src_ref=buf, dst_ref=buf,
                send_sem=ssem.at[sem_i], recv_sem=rsem.at[sem_i],
                device_id=(dev,), device_id_type=pl.DeviceIdType.MESH,
            )

        for phase in PHASES:
            for chunk, src, dst in phase:
                kbuf, vbuf = bufs[chunk]

                @pl.when(my == src)
                def _(kbuf=kbuf, vbuf=vbuf, chunk=chunk, dst=dst):
                    rk = mk(kbuf, 2 * chunk, dst)
                    rv = mk(vbuf, 2 * chunk + 1, dst)
                    rk.start()
                    rv.start()
                    rk.wait_send()
                    rv.wait_send()

                @pl.when(my == dst)
                def _(kbuf=kbuf, vbuf=vbuf, chunk=chunk, src=src):
                    rk = mk(kbuf, 2 * chunk, src)
                    rv = mk(vbuf, 2 * chunk + 1, src)
                    rk.wait_recv()
                    rv.wait_recv()

    def _do_compute(x_ref, wq_ref, wo_ref, out_ref, k0, v0, k1, v1):
        qi = lax.broadcasted_iota(jnp.int32, (Sq, KV_KEEP), 0)
        ki = lax.broadcasted_iota(jnp.int32, (Sq, KV_KEEP), 1)
        mask = jnp.abs(qi - ki) <= WIN

        wq = wq_ref[...].astype(jnp.bfloat16)
        wo = wo_ref[...].astype(jnp.bfloat16)

        for b in range(B):
            xb = x_ref[b].astype(jnp.bfloat16)
            qb = jnp.dot(
                xb, wq, preferred_element_type=jnp.float32
            ).astype(jnp.bfloat16)
            heads = []
            for h in range(Hq):
                q = qb[:, h * Dh:(h + 1) * Dh]
                kc = jnp.concatenate(
                    [k0[b, :, h, :], k1[b, :, h, :]], axis=0
                )
                vc = jnp.concatenate(
                    [v0[b, :, h, :], v1[b, :, h, :]], axis=0
                )
                s = lax.dot_general(
                    q, kc, (((1,), (1,)), ((), ())),
                    preferred_element_type=jnp.float32,
                ) * 0.125
                s = jnp.where(mask, s, -1e9)
                m = jnp.max(s, axis=1, keepdims=True)
                p = jnp.exp(s - m)
                l = jnp.sum(p, axis=1, keepdims=True)
                pn = (p / l).astype(jnp.bfloat16)
                ctx = lax.dot_general(
                    pn, vc, (((1,), (0,)), ((), ())),
                    preferred_element_type=jnp.float32,
                )
                heads.append(ctx.astype(jnp.bfloat16))
            ctx_b = jnp.concatenate(heads, axis=1)
            out_ref[b] = jnp.dot(
                ctx_b, wo, preferred_element_type=jnp.float32
            )

    def _do_exit_barrier(my):
        @functools.partial(
            pl.run_scoped, second_barrier=pltpu.SemaphoreType.REGULAR
        )
        def _(second_barrier):
            for d in range(N_DEV):
                @pl.when(my == d)
                def _(d=d):
                    for p in PEERS[d]:
                        pl.semaphore_signal(
                            second_barrier, inc=1, device_id=(p,),
                            device_id_type=pl.DeviceIdType.MESH,
                        )
                    pl.semaphore_wait(second_barrier, len(PEERS[d]))

    out_shape = jax.ShapeDtypeStruct((B, Sq, Dout), jnp.float32)
    return pl.pallas_call(
        body,
        out_shape=out_shape,
        in_specs=[pl.BlockSpec(memory_space=pltpu.VMEM)] * 5,
        out_specs=pl.BlockSpec(memory_space=pltpu.VMEM),
        scratch_shapes=[
            pltpu.VMEM((B, Skv_sh, Hq, Dh), jnp.bfloat16),
            pltpu.VMEM((B, Skv_sh, Hq, Dh), jnp.bfloat16),
            pltpu.VMEM((B, half, Hq, Dh), jnp.bfloat16),
            pltpu.VMEM((B, half, Hq, Dh), jnp.bfloat16),
            pltpu.SemaphoreType.DMA((4,)),
            pltpu.SemaphoreType.DMA((4,)),
        ],
        compiler_params=pltpu.CompilerParams(
            collective_id=None if _SKIP_COMM else 0
        ),
    )(x, Wq, K_ext, V_ext, Wo)


# device time: 36267 ns/iter; 1.6383x vs baseline; 1.4596x over previous
import os

import jax
import jax.numpy as jnp
from jax import lax
from jax.experimental import pallas as pl
from jax.experimental.pallas import tpu as pltpu

N_DEV = 8
WIN = 128
KV_KEEP = 384

try:
    _FLAGS = set(
        open(os.path.join(os.path.dirname(__file__), "exp_flags.txt"))
        .read()
        .split()
    )
except OSError:
    _FLAGS = set()
_SKIP_COMM = "skip_comm" in _FLAGS or os.environ.get("KERNEL_SKIP_COMM") == "1"
_SKIP_COMPUTE = (
    "skip_compute" in _FLAGS or os.environ.get("KERNEL_SKIP_COMPUTE") == "1"
)

SCHEDULE = {
    0: [("send", 0, [1, 3, 4]), ("recv", 1, 1), ("send", 1, [7])],
    1: [("send", 1, [0, 2, 5]), ("recv", 0, 0), ("send", 0, [6])],
    2: [("recv", 1, 1), ("send", 1, [3]), ("recv", 0, 3)],
    3: [("recv", 0, 0), ("send", 0, [2]), ("recv", 1, 2)],
    4: [("recv", 0, 0), ("send", 0, [5, 7]), ("recv", 1, 5)],
    5: [("recv", 1, 1), ("send", 1, [4, 6]), ("recv", 0, 4)],
    6: [("recv", 0, 1), ("recv", 1, 5)],
    7: [("recv", 0, 4), ("recv", 1, 0)],
}

WRITERS = {d: [] for d in range(N_DEV)}
N_WAITS = {d: 0 for d in range(N_DEV)}
for _d, _prog in SCHEDULE.items():
    for _step in _prog:
        if _step[0] == "send":
            N_WAITS[_d] += len(_step[2])
            for _t in _step[2]:
                WRITERS[_t].append(_d)


def kernel(x, Wq, K_ext, V_ext, Wo):
    B, Sq, Dm = x.shape
    _, Skv_sh, Hq, Dh = K_ext.shape
    Dout = Wo.shape[1]
    half = Skv_sh // 2

    def body(x_ref, wq_ref, k_ref, v_ref, wo_ref, out_ref,
             k0, v0, k1, v1, ssem, rsem):
        my = lax.axis_index("i")
        if not _SKIP_COMM:
            _do_comm(my, k_ref, v_ref, k0, v0, k1, v1, ssem, rsem)
        if not _SKIP_COMPUTE:
            _do_compute(x_ref, wq_ref, wo_ref, out_ref, k0, v0, k1, v1)

    def _do_comm(my, k_ref, v_ref, k0, v0, k1, v1, ssem, rsem):
        barrier = pltpu.get_barrier_semaphore()
        bufs = {0: (k0, v0), 1: (k1, v1)}

        def mk(buf, buf_i, slot, dev):
            return pltpu.make_async_remote_copy(
                src_ref=buf, dst_ref=buf,
                send_sem=ssem.at[buf_i, slot], recv_sem=rsem.at[buf_i],
                device_id=(dev,), device_id_type=pl.DeviceIdType.MESH,
            )

        def prog(d):
            for w in WRITERS[d]:
                pl.semaphore_signal(
                    barrier, inc=1, device_id=(w,),
                    device_id_type=pl.DeviceIdType.MESH,
                )
            if d == 0:
                k0[...] = k_ref[...].astype(jnp.bfloat16)
                v0[...] = v_ref[...].astype(jnp.bfloat16)
            if d == 1:
                k1[...] = k_ref[:, :half].astype(jnp.bfloat16)
                v1[...] = v_ref[:, :half].astype(jnp.bfloat16)
            if N_WAITS[d]:
                pl.semaphore_wait(barrier, N_WAITS[d])

            started = []
            slots = {0: 0, 1: 0}
            for step in SCHEDULE[d]:
                if step[0] == "send":
                    _, chunk, dsts = step
                    kbuf, vbuf = bufs[chunk]
                    for dst in dsts:
                        s = slots[chunk]
                        slots[chunk] = s + 1
                        rk = mk(kbuf, 2 * chunk, s, dst)
                        rv = mk(vbuf, 2 * chunk + 1, s, dst)
                        rk.start()
                        rv.start()
                        started += [rk, rv]
                else:
                    _, chunk, src = step
                    kbuf, vbuf = bufs[chunk]
                    mk(kbuf, 2 * chunk, 0, src).wait_recv()
                    mk(vbuf, 2 * chunk + 1, 0, src).wait_recv()
            for r in started:
                r.wait_send()

        for d in range(N_DEV):
            @pl.when(my == d)
            def _(d=d):
                prog(d)

    def _do_compute(x_ref, wq_ref, wo_ref, out_ref, k0, v0, k1, v1):
        qi = lax.broadcasted_iota(jnp.int32, (Sq, KV_KEEP), 0)
        ki = lax.broadcasted_iota(jnp.int32, (Sq, KV_KEEP), 1)
        mask = jnp.abs(qi - ki) <= WIN

        wq = wq_ref[...].astype(jnp.bfloat16)
        wo = wo_ref[...].astype(jnp.bfloat16)

        for b in range(B):
            xb = x_ref[b].astype(jnp.bfloat16)
            qb = jnp.dot(
                xb, wq, preferred_element_type=jnp.float32
            ).astype(jnp.bfloat16)
            heads = []
            for h in range(Hq):
                q = qb[:, h * Dh:(h + 1) * Dh]
                kc = jnp.concatenate(
                    [k0[b, :, h, :], k1[b, :, h, :]], axis=0
                )
                vc = jnp.concatenate(
                    [v0[b, :, h, :], v1[b, :, h, :]], axis=0
                )
                s = lax.dot_general(
                    q, kc, (((1,), (1,)), ((), ())),
                    preferred_element_type=jnp.float32,
                )
                p = jnp.where(mask, jnp.exp(s * 0.125), 0.0)
                l = jnp.sum(p, axis=1, keepdims=True)
                ctx = lax.dot_general(
                    p.astype(jnp.bfloat16), vc, (((1,), (0,)), ((), ())),
                    preferred_element_type=jnp.float32,
                ) / l
                heads.append(ctx.astype(jnp.bfloat16))
            ctx_b = jnp.concatenate(heads, axis=1)
            out_ref[b] = jnp.dot(
                ctx_b, wo, preferred_element_type=jnp.float32
            )

    out_shape = jax.ShapeDtypeStruct((B, Sq, Dout), jnp.float32)
    return pl.pallas_call(
        body,
        out_shape=out_shape,
        in_specs=[pl.BlockSpec(memory_space=pltpu.VMEM)] * 5,
        out_specs=pl.BlockSpec(memory_space=pltpu.VMEM),
        scratch_shapes=[
            pltpu.VMEM((B, Skv_sh, Hq, Dh), jnp.bfloat16),
            pltpu.VMEM((B, Skv_sh, Hq, Dh), jnp.bfloat16),
            pltpu.VMEM((B, half, Hq, Dh), jnp.bfloat16),
            pltpu.VMEM((B, half, Hq, Dh), jnp.bfloat16),
            pltpu.SemaphoreType.DMA((4, 3)),
            pltpu.SemaphoreType.DMA((4,)),
        ],
        compiler_params=pltpu.CompilerParams(
            collective_id=None if _SKIP_COMM else 0
        ),
    )(x, Wq, K_ext, V_ext, Wo)
